# Initial kernel scaffold; baseline (speedup 1.0000x reference)
import jax
import jax.numpy as jnp
from jax.experimental import pallas as pl


def kernel(betas, t):
    raise NotImplementedError("write your pallas kernel here")


# speedup vs baseline: 20.1575x; 20.1575x over previous
import functools

import jax
import jax.numpy as jnp
from jax import lax
from jax.experimental import pallas as pl
from jax.experimental.pallas import tpu as pltpu
from jax.experimental.pallas import tpu_sc as plsc

N = 1001
NPAD = 1024
B = 16384
NBUF = 6
NW = 32
BPW = B // NW
L = 16


def _table_body(betas_ref, out_ref):
    b = betas_ref[...]
    a = 1.0 - b
    la = jnp.log(a)
    ii = lax.broadcasted_iota(jnp.int32, (NPAD, NPAD), 0)
    jj = lax.broadcasted_iota(jnp.int32, (NPAD, NPAD), 1)
    tri = (ii <= jj).astype(jnp.float32)
    cs = lax.dot_general(la, tri, (((1,), (0,)), ((), ())),
                         precision=lax.Precision.HIGHEST)
    ab = jnp.exp(cs)
    out_ref[0:1, :] = b
    out_ref[1:2, :] = jnp.sqrt(b)
    out_ref[2:3, :] = ab
    out_ref[3:4, :] = jnp.exp(0.5 * cs)
    out_ref[4:5, :] = jnp.sqrt(jnp.maximum(1.0 - ab, 0.0))
    out_ref[5:6, :] = lax.rsqrt(a)


def _build_table(betas_padded):
    return pl.pallas_call(
        _table_body,
        out_shape=jax.ShapeDtypeStruct((NBUF, NPAD), jnp.float32),
    )(betas_padded)


@functools.cache
def _make_gather_kernel():
    mesh = plsc.VectorSubcoreMesh(core_axis_name="c", subcore_axis_name="s")

    @functools.partial(
        pl.kernel,
        out_type=jax.ShapeDtypeStruct((NBUF, B), jnp.float32),
        mesh=mesh,
        compiler_params=pltpu.CompilerParams(needs_layout_passes=False),
        scratch_types=[
            pltpu.VMEM((NBUF * NPAD,), jnp.float32),
            pltpu.VMEM((BPW,), jnp.int32),
            pltpu.VMEM((NBUF, BPW), jnp.float32),
        ],
    )
    def _gather_kernel(table_hbm, t_hbm, out_hbm, table_v, t_v, out_v):
        wid = lax.axis_index("s") * 2 + lax.axis_index("c")
        base = wid * BPW
        pltpu.sync_copy(table_hbm, table_v)
        pltpu.sync_copy(t_hbm.at[pl.ds(base, BPW)], t_v)

        def body(i, carry):
            tv = t_v[pl.ds(i * L, L)]
            for j in range(NBUF):
                vals = plsc.load_gather(table_v, [tv + (j * NPAD)])
                out_v[j, pl.ds(i * L, L)] = vals
            return carry

        lax.fori_loop(0, BPW // L, body, 0)
        for j in range(NBUF):
            pltpu.sync_copy(out_v.at[j], out_hbm.at[j, pl.ds(base, BPW)])

    return _gather_kernel


def kernel(betas, t):
    betas_p = jnp.zeros((1, NPAD), jnp.float32).at[0, :N].set(betas)
    table = _build_table(betas_p).reshape(NBUF * NPAD)
    out = _make_gather_kernel()(table, t.astype(jnp.int32))
    return out.reshape(NBUF, B, 1, 1, 1)


# speedup vs baseline: 20.8315x; 1.0334x over previous
import functools

import jax
import jax.numpy as jnp
from jax import lax
from jax.experimental import pallas as pl
from jax.experimental.pallas import tpu as pltpu
from jax.experimental.pallas import tpu_sc as plsc

N = 1001
NPAD = 1024
B = 16384
NBUF = 6
NW = 32
BPW = B // NW
L = 16
CHUNK = NPAD // 16
XCH = NBUF * NPAD
XLEN = XCH + 16 * L


def _rsqrt(x):
    i = plsc.bitcast(x, jnp.int32)
    i = 0x5F3759DF - (i >> 1)
    y = plsc.bitcast(i, jnp.float32)
    for _ in range(3):
        y = y * (1.5 - 0.5 * x * y * y)
    return y


def _vscan_mul(x, scr, iota):
    for sh in (1, 2, 4, 8):
        scr[...] = x
        shifted = plsc.load_gather(scr, [jnp.maximum(iota - sh, 0)])
        x = jnp.where(iota >= sh, x * shifted, x)
    return x


def _bcast_lane(x, k, scr):
    scr[...] = x
    return plsc.load_gather(scr, [jnp.full((L,), k, jnp.int32)])


@functools.cache
def _make_kernel():
    mesh = plsc.VectorSubcoreMesh(core_axis_name="c", subcore_axis_name="s")

    @functools.partial(
        pl.kernel,
        out_type=(
            jax.ShapeDtypeStruct((NBUF, B), jnp.float32),
            jax.ShapeDtypeStruct((2, XLEN), jnp.float32),
        ),
        mesh=mesh,
        compiler_params=pltpu.CompilerParams(needs_layout_passes=False),
        scratch_types=[
            pltpu.VMEM((CHUNK,), jnp.float32),
            pltpu.VMEM((L,), jnp.float32),
            pltpu.VMEM((L,), jnp.float32),
            pltpu.VMEM((16 * L,), jnp.float32),
            pltpu.VMEM((NBUF * CHUNK,), jnp.float32),
            pltpu.VMEM((NBUF * NPAD,), jnp.float32),
            pltpu.VMEM((BPW,), jnp.int32),
            pltpu.VMEM((NBUF, BPW), jnp.float32),
        ],
    )
    def _sched_kernel(betas_hbm, t_hbm, out_hbm, xch_hbm,
                      bchunk, scr, pstage, pall, rowchunk,
                      table_v, t_v, out_v):
        c = lax.axis_index("c")
        s = lax.axis_index("s")
        iota = lax.broadcasted_iota(jnp.int32, (L,), 0)

        base = s * CHUNK
        pltpu.sync_copy(betas_hbm.at[pl.ds(base, CHUNK)], bchunk)

        a_scans = []
        b_vecs = []
        a_vecs = []
        carry = jnp.full((L,), 1.0, jnp.float32)
        for k in range(CHUNK // L):
            b_k = bchunk[pl.ds(k * L, L)]
            a_k = 1.0 - b_k
            sk = _vscan_mul(a_k, scr, iota) * carry
            carry = _bcast_lane(sk, L - 1, scr)
            b_vecs.append(b_k)
            a_vecs.append(a_k)
            a_scans.append(sk)

        pstage[...] = carry
        pltpu.sync_copy(pstage, xch_hbm.at[c, pl.ds(XCH + s * L, L)])
        plsc.subcore_barrier()
        pltpu.sync_copy(xch_hbm.at[c, pl.ds(XCH, 16 * L)], pall)
        pvals = plsc.load_gather(pall, [iota * L])
        pscan = _vscan_mul(pvals, scr, iota)
        prev = _bcast_lane(pscan, jnp.maximum(s - 1, 0), scr)
        pre = jnp.where(jnp.full((L,), s, jnp.int32) == 0,
                        jnp.full((L,), 1.0, jnp.float32), prev)

        for k in range(CHUNK // L):
            b_k = b_vecs[k]
            ab_k = a_scans[k] * pre
            rb = _rsqrt(jnp.maximum(b_k, 1e-30))
            rab = _rsqrt(ab_k)
            romab = _rsqrt(jnp.maximum(1.0 - ab_k, 1e-30))
            off = k * L
            rowchunk[pl.ds(0 * CHUNK + off, L)] = b_k
            rowchunk[pl.ds(1 * CHUNK + off, L)] = b_k * rb
            rowchunk[pl.ds(2 * CHUNK + off, L)] = ab_k
            rowchunk[pl.ds(3 * CHUNK + off, L)] = ab_k * rab
            rowchunk[pl.ds(4 * CHUNK + off, L)] = (1.0 - ab_k) * romab
            rowchunk[pl.ds(5 * CHUNK + off, L)] = _rsqrt(a_vecs[k])

        for j in range(NBUF):
            pltpu.sync_copy(rowchunk.at[pl.ds(j * CHUNK, CHUNK)],
                            xch_hbm.at[c, pl.ds(j * NPAD + base, CHUNK)])
        plsc.subcore_barrier()

        wid = s * 2 + c
        tbase = wid * BPW
        pltpu.sync_copy(xch_hbm.at[c, pl.ds(0, NBUF * NPAD)], table_v)
        pltpu.sync_copy(t_hbm.at[pl.ds(tbase, BPW)], t_v)

        def body(i, carry2):
            tv = t_v[pl.ds(i * L, L)]
            for j in range(NBUF):
                vals = plsc.load_gather(table_v, [tv + (j * NPAD)])
                out_v[j, pl.ds(i * L, L)] = vals
            return carry2

        lax.fori_loop(0, BPW // L, body, 0)
        for j in range(NBUF):
            pltpu.sync_copy(out_v.at[j], out_hbm.at[j, pl.ds(tbase, BPW)])

    return _sched_kernel


def kernel(betas, t):
    betas_p = jnp.zeros((NPAD,), jnp.float32).at[:N].set(betas)
    out, _ = _make_kernel()(betas_p, t.astype(jnp.int32))
    return out.reshape(NBUF, B, 1, 1, 1)


# speedup vs baseline: 22.1825x; 1.0649x over previous
import functools

import jax
import jax.numpy as jnp
from jax import lax
from jax.experimental import pallas as pl
from jax.experimental.pallas import tpu as pltpu
from jax.experimental.pallas import tpu_sc as plsc

N = 1001
NPAD = 1024
B = 16384
NBUF = 6
NW = 32
BPW = B // NW
L = 16
CHUNK = NPAD // 16
SLOTS = 8
XCH = NPAD * SLOTS
XLEN = XCH + 16 * L
TAIL = N - 15 * CHUNK


def _rsqrt(x):
    i = plsc.bitcast(x, jnp.int32)
    i = 0x5F3759DF - (i >> 1)
    y = plsc.bitcast(i, jnp.float32)
    for _ in range(3):
        y = y * (1.5 - 0.5 * x * y * y)
    return y


def _vscan_mul(x, scr, iota):
    for sh in (1, 2, 4, 8):
        scr[...] = x
        shifted = plsc.load_gather(scr, [jnp.maximum(iota - sh, 0)])
        x = jnp.where(iota >= sh, x * shifted, x)
    return x


def _bcast_lane(x, k, scr):
    scr[...] = x
    return plsc.load_gather(scr, [jnp.full((L,), k, jnp.int32)])


@functools.cache
def _make_kernel():
    mesh = plsc.VectorSubcoreMesh(core_axis_name="c", subcore_axis_name="s")

    @functools.partial(
        pl.kernel,
        out_type=(
            jax.ShapeDtypeStruct((NBUF * B,), jnp.float32),
            jax.ShapeDtypeStruct((2, XLEN), jnp.float32),
        ),
        mesh=mesh,
        compiler_params=pltpu.CompilerParams(
            needs_layout_passes=False, skip_device_barrier=True),
        scratch_types=[
            pltpu.VMEM((CHUNK,), jnp.float32),
            pltpu.VMEM((L,), jnp.float32),
            pltpu.VMEM((L,), jnp.float32),
            pltpu.VMEM((16 * L,), jnp.float32),
            pltpu.VMEM((CHUNK * SLOTS,), jnp.float32),
            pltpu.VMEM((NPAD * SLOTS,), jnp.float32),
            pltpu.VMEM((BPW,), jnp.int32),
            pltpu.VMEM((NBUF, BPW), jnp.float32),
            pltpu.SemaphoreType.DMA,
            pltpu.SemaphoreType.DMA,
        ],
    )
    def _sched_kernel(betas_hbm, t_hbm, out_hbm, xch_hbm,
                      bchunk, scr, pstage, pall, rowchunk,
                      table_v, t_v, out_v, sem_t, sem_o):
        c = lax.axis_index("c")
        s = lax.axis_index("s")
        iota = lax.broadcasted_iota(jnp.int32, (L,), 0)
        wid = s * 2 + c
        tbase = wid * BPW

        tcopy = pltpu.async_copy(t_hbm.at[pl.ds(tbase, BPW)], t_v, sem_t)

        base = s * CHUNK

        @pl.when(s < 15)
        def _():
            pltpu.sync_copy(betas_hbm.at[pl.ds(base, CHUNK)], bchunk)

        @pl.when(s == 15)
        def _():
            pltpu.sync_copy(betas_hbm.at[pl.ds(15 * CHUNK, TAIL)],
                            bchunk.at[pl.ds(0, TAIL)])

        b_vecs = []
        a_vecs = []
        a_scans = []
        carry = jnp.full((L,), 1.0, jnp.float32)
        for k in range(CHUNK // L):
            g = base + (k * L) + iota
            raw = bchunk[pl.ds(k * L, L)]
            b_k = jnp.where(g <= N - 1, raw, 0.0)
            a_k = 1.0 - b_k
            sk = _vscan_mul(a_k, scr, iota) * carry
            carry = _bcast_lane(sk, L - 1, scr)
            b_vecs.append(b_k)
            a_vecs.append(a_k)
            a_scans.append(sk)

        pstage[...] = carry
        pltpu.sync_copy(pstage, xch_hbm.at[c, pl.ds(XCH + s * L, L)])
        plsc.subcore_barrier()
        pltpu.sync_copy(xch_hbm.at[c, pl.ds(XCH, 16 * L)], pall)
        pvals = plsc.load_gather(pall, [iota * L])
        pscan = _vscan_mul(pvals, scr, iota)
        prev = _bcast_lane(pscan, jnp.maximum(s - 1, 0), scr)
        pre = jnp.where(jnp.full((L,), s, jnp.int32) == 0, 1.0, prev)

        for k in range(CHUNK // L):
            b_k = b_vecs[k]
            ab_k = a_scans[k] * pre
            rb = _rsqrt(jnp.maximum(b_k, 1e-30))
            rab = _rsqrt(ab_k)
            omab = 1.0 - ab_k
            romab = _rsqrt(jnp.maximum(omab, 1e-30))
            vals6 = (b_k, b_k * rb, ab_k, ab_k * rab, omab * romab,
                     _rsqrt(a_vecs[k]))
            idx8 = (iota + k * L) * SLOTS
            for j, v in enumerate(vals6):
                plsc.store_scatter(rowchunk, [idx8 + j], v)
        pltpu.sync_copy(rowchunk,
                        xch_hbm.at[c, pl.ds(base * SLOTS, CHUNK * SLOTS)])
        plsc.subcore_barrier()

        pltpu.sync_copy(xch_hbm.at[c, pl.ds(0, NPAD * SLOTS)], table_v)
        tcopy.wait()
        for i in range(BPW // L):
            t8 = t_v[pl.ds(i * L, L)] * SLOTS
            for j in range(NBUF):
                vals = plsc.load_gather(table_v, [t8 + j])
                out_v[j, pl.ds(i * L, L)] = vals
        copies = [
            pltpu.async_copy(out_v.at[j],
                             out_hbm.at[pl.ds(j * B + tbase, BPW)],
                             sem_o)
            for j in range(NBUF)
        ]
        for cp in copies:
            cp.wait()

    return _sched_kernel


def kernel(betas, t):
    out, _ = _make_kernel()(betas, t.astype(jnp.int32))
    return out.reshape(NBUF, B, 1, 1, 1)


# speedup vs baseline: 23.1412x; 1.0432x over previous
import functools

import jax
import jax.numpy as jnp
from jax import lax
from jax.experimental import pallas as pl
from jax.experimental.pallas import tpu as pltpu
from jax.experimental.pallas import tpu_sc as plsc

N = 1001
NPAD = 1024
B = 16384
NBUF = 6
NW = 32
BPW = B // NW
L = 16
CHUNK = NPAD // 16
SLOTS = 8
XCH = NPAD * SLOTS
XLEN = XCH + 16 * L
TAIL = N - 15 * CHUNK


def _rsqrt(x):
    i = plsc.bitcast(x, jnp.int32)
    i = 0x5F3759DF - (i >> 1)
    y = plsc.bitcast(i, jnp.float32)
    for _ in range(3):
        y = y * (1.5 - 0.5 * x * y * y)
    return y


def _vscan_mul(x, scr, iota):
    for sh in (1, 2, 4, 8):
        scr[...] = x
        shifted = plsc.load_gather(scr, [jnp.maximum(iota - sh, 0)])
        x = jnp.where(iota >= sh, x * shifted, x)
    return x


def _bcast_lane(x, k, scr):
    scr[...] = x
    return plsc.load_gather(scr, [jnp.full((L,), k, jnp.int32)])


@functools.cache
def _make_kernel():
    mesh = plsc.VectorSubcoreMesh(core_axis_name="c", subcore_axis_name="s")

    @functools.partial(
        pl.kernel,
        out_type=(
            jax.ShapeDtypeStruct((NBUF * B,), jnp.float32),
            jax.ShapeDtypeStruct((2, XLEN), jnp.float32),
        ),
        mesh=mesh,
        compiler_params=pltpu.CompilerParams(
            needs_layout_passes=False, skip_device_barrier=True),
        scratch_types=[
            pltpu.VMEM((CHUNK,), jnp.float32),
            pltpu.VMEM((L,), jnp.float32),
            pltpu.VMEM((L,), jnp.float32),
            pltpu.VMEM((16 * L,), jnp.float32),
            pltpu.VMEM((CHUNK * SLOTS,), jnp.float32),
            pltpu.VMEM((NPAD * SLOTS,), jnp.float32),
            pltpu.VMEM((BPW,), jnp.int32),
            pltpu.VMEM((NBUF, BPW), jnp.float32),
            pltpu.SemaphoreType.DMA,
            pltpu.SemaphoreType.DMA,
        ],
    )
    def _sched_kernel(betas_hbm, t_hbm, out_hbm, xch_hbm,
                      bchunk, scr, pstage, pall, rowchunk,
                      table_v, t_v, out_v, sem_t, sem_o):
        c = lax.axis_index("c")
        s = lax.axis_index("s")
        iota = lax.broadcasted_iota(jnp.int32, (L,), 0)
        wid = s * 2 + c
        tbase = wid * BPW

        tcopy = pltpu.async_copy(t_hbm.at[pl.ds(tbase, BPW)], t_v, sem_t)

        base = s * CHUNK

        @pl.when(s < 15)
        def _():
            pltpu.sync_copy(betas_hbm.at[pl.ds(base, CHUNK)], bchunk)

        @pl.when(s == 15)
        def _():
            pltpu.sync_copy(betas_hbm.at[pl.ds(15 * CHUNK, TAIL)],
                            bchunk.at[pl.ds(0, TAIL)])

        b_vecs = []
        a_vecs = []
        a_scans = []
        carry = jnp.full((L,), 1.0, jnp.float32)
        for k in range(CHUNK // L):
            g = base + (k * L) + iota
            raw = bchunk[pl.ds(k * L, L)]
            b_k = jnp.where(g <= N - 1, raw, 0.0)
            a_k = 1.0 - b_k
            sk = _vscan_mul(a_k, scr, iota) * carry
            carry = _bcast_lane(sk, L - 1, scr)
            b_vecs.append(b_k)
            a_vecs.append(a_k)
            a_scans.append(sk)

        pstage[...] = carry
        pltpu.sync_copy(pstage, xch_hbm.at[c, pl.ds(XCH + s * L, L)])
        plsc.subcore_barrier()
        pltpu.sync_copy(xch_hbm.at[c, pl.ds(XCH, 16 * L)], pall)
        pvals = plsc.load_gather(pall, [iota * L])
        pscan = _vscan_mul(pvals, scr, iota)
        prev = _bcast_lane(pscan, jnp.maximum(s - 1, 0), scr)
        pre = jnp.where(jnp.full((L,), s, jnp.int32) == 0, 1.0, prev)

        for k in range(CHUNK // L):
            b_k = b_vecs[k]
            ab_k = a_scans[k] * pre
            rb = _rsqrt(jnp.maximum(b_k, 1e-30))
            rab = _rsqrt(ab_k)
            omab = 1.0 - ab_k
            romab = _rsqrt(jnp.maximum(omab, 1e-30))
            vals6 = (b_k, b_k * rb, ab_k, ab_k * rab, omab * romab,
                     _rsqrt(a_vecs[k]))
            idx8 = (iota + k * L) * SLOTS
            for j, v in enumerate(vals6):
                plsc.store_scatter(rowchunk, [idx8 + j], v)
        pltpu.sync_copy(rowchunk,
                        xch_hbm.at[c, pl.ds(base * SLOTS, CHUNK * SLOTS)])
        plsc.subcore_barrier()

        pltpu.sync_copy(xch_hbm.at[c, pl.ds(0, NPAD * SLOTS)], table_v)
        tcopy.wait()

        def body(i, carry2):
            t8 = t_v[pl.ds(i * L, L)] * SLOTS
            for j in range(NBUF):
                vals = plsc.load_gather(table_v, [t8 + j])
                out_v[j, pl.ds(i * L, L)] = vals
            return carry2

        lax.fori_loop(0, BPW // L, body, 0)
        copies = [
            pltpu.async_copy(out_v.at[j],
                             out_hbm.at[pl.ds(j * B + tbase, BPW)],
                             sem_o)
            for j in range(NBUF)
        ]
        for cp in copies:
            cp.wait()

    return _sched_kernel


def kernel(betas, t):
    out, _ = _make_kernel()(betas, t.astype(jnp.int32))
    return out.reshape(NBUF, B, 1, 1, 1)


# speedup vs baseline: 24.0335x; 1.0386x over previous
import functools

import jax
import jax.numpy as jnp
from jax import lax
from jax.experimental import pallas as pl
from jax.experimental.pallas import tpu as pltpu
from jax.experimental.pallas import tpu_sc as plsc

N = 1001
NPAD = 1024
B = 16384
NBUF = 6
NW = 32
BPW = B // NW
L = 16
CHUNK = NPAD // 16
SLOTS = 8
XCH = NPAD * SLOTS
XLEN = XCH + 16 * L
TAIL = N - 15 * CHUNK


def _rsqrt(x):
    i = plsc.bitcast(x, jnp.int32)
    i = 0x5F3759DF - (i >> 1)
    y = plsc.bitcast(i, jnp.float32)
    for _ in range(3):
        y = y * (1.5 - 0.5 * x * y * y)
    return y


def _vscan_mul(x, scr, iota):
    for sh in (1, 2, 4, 8):
        scr[...] = x
        shifted = plsc.load_gather(scr, [jnp.maximum(iota - sh, 0)])
        x = jnp.where(iota >= sh, x * shifted, x)
    return x


def _bcast_lane(x, k, scr):
    scr[...] = x
    return plsc.load_gather(scr, [jnp.full((L,), k, jnp.int32)])


@functools.cache
def _make_kernel():
    mesh = plsc.VectorSubcoreMesh(core_axis_name="c", subcore_axis_name="s")

    @functools.partial(
        pl.kernel,
        out_type=(
            jax.ShapeDtypeStruct((NBUF * B,), jnp.float32),
            jax.ShapeDtypeStruct((2, XLEN), jnp.float32),
        ),
        mesh=mesh,
        compiler_params=pltpu.CompilerParams(
            needs_layout_passes=False, skip_device_barrier=True),
        scratch_types=[
            pltpu.VMEM((CHUNK,), jnp.float32),
            pltpu.VMEM((L,), jnp.float32),
            pltpu.VMEM((L,), jnp.float32),
            pltpu.VMEM((16 * L,), jnp.float32),
            pltpu.VMEM((CHUNK * SLOTS,), jnp.float32),
            pltpu.VMEM((NPAD * SLOTS,), jnp.float32),
            pltpu.VMEM((BPW,), jnp.int32),
            pltpu.VMEM((NBUF, BPW), jnp.float32),
            pltpu.SemaphoreType.DMA,
            pltpu.SemaphoreType.DMA,
        ],
    )
    def _sched_kernel(betas_hbm, t_hbm, out_hbm, xch_hbm,
                      bchunk, scr, pstage, pall, rowchunk,
                      table_v, t_v, out_v, sem_t, sem_o):
        c = lax.axis_index("c")
        s = lax.axis_index("s")
        iota = lax.broadcasted_iota(jnp.int32, (L,), 0)
        wid = s * 2 + c
        tbase = wid * BPW

        tcopy = pltpu.async_copy(t_hbm.at[pl.ds(tbase, BPW)], t_v, sem_t)

        base = s * CHUNK

        @pl.when(s < 15)
        def _():
            pltpu.sync_copy(betas_hbm.at[pl.ds(base, CHUNK)], bchunk)

        @pl.when(s == 15)
        def _():
            pltpu.sync_copy(betas_hbm.at[pl.ds(15 * CHUNK, TAIL)],
                            bchunk.at[pl.ds(0, TAIL)])

        b_vecs = []
        a_vecs = []
        a_scans = []
        carry = jnp.full((L,), 1.0, jnp.float32)
        for k in range(CHUNK // L):
            g = base + (k * L) + iota
            raw = bchunk[pl.ds(k * L, L)]
            b_k = jnp.where(g <= N - 1, raw, 0.0)
            a_k = 1.0 - b_k
            sk = _vscan_mul(a_k, scr, iota) * carry
            carry = _bcast_lane(sk, L - 1, scr)
            b_vecs.append(b_k)
            a_vecs.append(a_k)
            a_scans.append(sk)

        pstage[...] = carry
        pltpu.sync_copy(pstage, xch_hbm.at[c, pl.ds(XCH + s * L, L)])
        plsc.subcore_barrier()
        pltpu.sync_copy(xch_hbm.at[c, pl.ds(XCH, 16 * L)], pall)
        pvals = plsc.load_gather(pall, [iota * L])
        pscan = _vscan_mul(pvals, scr, iota)
        prev = _bcast_lane(pscan, jnp.maximum(s - 1, 0), scr)
        pre = jnp.where(jnp.full((L,), s, jnp.int32) == 0, 1.0, prev)

        for k in range(CHUNK // L):
            b_k = b_vecs[k]
            ab_k = a_scans[k] * pre
            rb = _rsqrt(jnp.maximum(b_k, 1e-30))
            rab = _rsqrt(ab_k)
            omab = 1.0 - ab_k
            romab = _rsqrt(jnp.maximum(omab, 1e-30))
            vals6 = (b_k, b_k * rb, ab_k, ab_k * rab, omab * romab,
                     _rsqrt(a_vecs[k]))
            idx8 = (iota + k * L) * SLOTS
            for j, v in enumerate(vals6):
                plsc.store_scatter(rowchunk, [idx8 + j], v)
        pltpu.sync_copy(rowchunk,
                        xch_hbm.at[c, pl.ds(base * SLOTS, CHUNK * SLOTS)])
        plsc.subcore_barrier()

        pltpu.sync_copy(xch_hbm.at[c, pl.ds(0, NPAD * SLOTS)], table_v)
        tcopy.wait()

        @plsc.parallel_loop(0, BPW // L, unroll=4)
        def _gather_body(i):
            t8 = t_v[pl.ds(i * L, L)] * SLOTS
            for j in range(NBUF):
                vals = plsc.load_gather(table_v, [t8 + j])
                out_v[j, pl.ds(i * L, L)] = vals
        copies = [
            pltpu.async_copy(out_v.at[j],
                             out_hbm.at[pl.ds(j * B + tbase, BPW)],
                             sem_o)
            for j in range(NBUF)
        ]
        for cp in copies:
            cp.wait()

    return _sched_kernel


def kernel(betas, t):
    out, _ = _make_kernel()(betas, t.astype(jnp.int32))
    return out.reshape(NBUF, B, 1, 1, 1)
